# baseline (speedup 1.0000x reference)
import functools

import jax
import jax.numpy as jnp
from jax import lax
from jax.experimental import pallas as pl
from jax.experimental.pallas import tpu as pltpu
from jax.experimental.pallas import tpu_sc as plsc

NUM_EMB = 100000
SEM_IDS_DIM = 4
EMB_DIM = 64
N_SEM = 3
MAX_TAG = 1000
N_TAG = SEM_IDS_DIM - N_SEM
SEM_OFF = NUM_EMB * N_SEM
TOTAL_EMB = SEM_OFF + MAX_TAG * N_TAG + 1
PAD_IDX = TOTAL_EMB - 1
B, L = 4096, 200
LF = 4

NC = 2
NS = 16
NW = NC * NS
LANES = 16

CHUNK = 512
IDX_ROWS = CHUNK // 128
NB = 2

SEQ_N = B * L
FUT_N = B * LF
SEQ_PER_W = SEQ_N // NW
FUT_PER_W = FUT_N // NW
SEQ_CHUNKS = SEQ_PER_W // CHUNK


def _compute_indices_chunk(sem_v, tok_v, idx_v):
    for i in range(CHUNK // LANES):
        s = sem_v[pl.ds(i * LANES, LANES)]
        t = tok_v[pl.ds(i * LANES, LANES)]
        sem_c = jnp.minimum(jnp.maximum(s, 0), NUM_EMB - 1)
        tag_c = jnp.minimum(jnp.maximum(s, 0), MAX_TAG - 1)
        idx_sem = t * NUM_EMB + sem_c
        tag_layer = t - N_SEM
        idx_tag = jnp.where(
            tag_layer < N_TAG, SEM_OFF + tag_layer * MAX_TAG + tag_c, PAD_IDX
        )
        idx = jnp.where(t < N_SEM, idx_sem, idx_tag)
        idx_v[i // 8, pl.ds((i % 8) * LANES, LANES)] = idx


def _body(sem_seq, tok_seq, sem_fut, tok_fut, table,
          out_seq, out_fut, sem_v, tok_v, idx_v, rows_v,
          id_sems, g_sems, w_sems):
    wid = lax.axis_index("s") * NC + lax.axis_index("c")
    seq_base0 = wid * SEQ_PER_W
    fut_base = wid * FUT_PER_W

    def fire_ids(src_s, src_t, base, b):
        pltpu.async_copy(src_s.at[pl.ds(base, CHUNK)], sem_v.at[b],
                         id_sems.at[b])
        pltpu.async_copy(src_t.at[pl.ds(base, CHUNK)], tok_v.at[b],
                         id_sems.at[b])

    def wait_ids(b):
        pltpu.make_async_copy(sem_seq.at[pl.ds(0, CHUNK)], sem_v.at[b],
                              id_sems.at[b]).wait()
        pltpu.make_async_copy(tok_seq.at[pl.ds(0, CHUNK)], tok_v.at[b],
                              id_sems.at[b]).wait()

    def fire_gathers(b):
        for j in range(IDX_ROWS):
            pltpu.async_copy(table.at[idx_v.at[b, j]],
                             rows_v.at[b, pl.ds(j * 128, 128)],
                             g_sems.at[b])

    def wait_gathers(b):
        for j in range(IDX_ROWS):
            pltpu.make_async_copy(table.at[idx_v.at[b, j]],
                                  rows_v.at[b, pl.ds(j * 128, 128)],
                                  g_sems.at[b]).wait()

    def fire_write(out_ref, base, b):
        pltpu.async_copy(rows_v.at[b], out_ref.at[pl.ds(base, CHUNK)],
                         w_sems.at[b])

    def wait_write(b):
        pltpu.make_async_copy(rows_v.at[b], out_seq.at[pl.ds(0, CHUNK)],
                              w_sems.at[b]).wait()

    def compute(b):
        _compute_indices_chunk(sem_v.at[b], tok_v.at[b], idx_v.at[b])

    sbase = lambda i: seq_base0 + i * CHUNK

    fire_ids(sem_seq, tok_seq, sbase(0), 0)
    fire_ids(sem_seq, tok_seq, sbase(1), 1)

    wait_ids(0)
    compute(0)
    fire_ids(sem_seq, tok_seq, sbase(2), 0)
    fire_gathers(0)

    wait_ids(1)
    compute(1)
    fire_ids(sem_seq, tok_seq, sbase(3), 1)
    fire_gathers(1)
    wait_gathers(0)
    fire_write(out_seq, sbase(0), 0)

    def loop_body(g, carry):
        for b in (0, 1):
            i = 2 * g + b
            nb = 1 - b
            wait_ids(b)
            compute(b)
            fire_ids(sem_seq, tok_seq, sbase(i + 2), b)
            wait_write(b)
            fire_gathers(b)
            wait_gathers(nb)
            fire_write(out_seq, sbase(i - 1), nb)
        return carry

    lax.fori_loop(1, SEQ_CHUNKS // 2 - 1, loop_body, 0)

    wait_ids(0)
    compute(0)
    fire_ids(sem_fut, tok_fut, fut_base, 0)
    wait_write(0)
    fire_gathers(0)
    wait_gathers(1)
    fire_write(out_seq, sbase(SEQ_CHUNKS - 3), 1)

    wait_ids(1)
    compute(1)
    wait_write(1)
    fire_gathers(1)
    wait_gathers(0)
    fire_write(out_seq, sbase(SEQ_CHUNKS - 2), 0)

    wait_ids(0)
    compute(0)
    wait_write(0)
    fire_gathers(0)
    wait_gathers(1)
    fire_write(out_seq, sbase(SEQ_CHUNKS - 1), 1)

    wait_gathers(0)
    fire_write(out_fut, fut_base, 0)
    wait_write(1)
    wait_write(0)


@jax.jit
def _emb_lookup(sem_seq, tok_seq, sem_fut, tok_fut, table):
    mesh = plsc.VectorSubcoreMesh(core_axis_name="c", subcore_axis_name="s")
    f = pl.kernel(
        _body,
        out_type=(
            jax.ShapeDtypeStruct((SEQ_N, EMB_DIM), jnp.float32),
            jax.ShapeDtypeStruct((FUT_N, EMB_DIM), jnp.float32),
        ),
        mesh=mesh,
        scratch_types=[
            pltpu.VMEM((NB, CHUNK), jnp.int32),
            pltpu.VMEM((NB, CHUNK), jnp.int32),
            pltpu.VMEM((NB, IDX_ROWS, 128), jnp.int32),
            pltpu.VMEM((NB, CHUNK, EMB_DIM), jnp.float32),
            pltpu.SemaphoreType.DMA((NB,)),
            pltpu.SemaphoreType.DMA((NB,)),
            pltpu.SemaphoreType.DMA((NB,)),
        ],
        compiler_params=pltpu.CompilerParams(use_tc_tiling_on_sc=False),
    )
    return f(sem_seq, tok_seq, sem_fut, tok_fut, table)


def kernel(sem_ids, token_type_ids, sem_ids_fut, token_type_ids_fut, emb_table):
    out_seq, out_fut = _emb_lookup(
        sem_ids.reshape(-1),
        token_type_ids.reshape(-1),
        sem_ids_fut.reshape(-1),
        token_type_ids_fut.reshape(-1),
        emb_table,
    )
    return (
        out_seq.reshape(B, L, EMB_DIM),
        out_fut.reshape(B, LF, EMB_DIM),
    )


# speedup vs baseline: 1.4710x; 1.4710x over previous
import functools

import jax
import jax.numpy as jnp
from jax import lax
from jax.experimental import pallas as pl
from jax.experimental.pallas import tpu as pltpu
from jax.experimental.pallas import tpu_sc as plsc

NUM_EMB = 100000
SEM_IDS_DIM = 4
EMB_DIM = 64
N_SEM = 3
MAX_TAG = 1000
N_TAG = SEM_IDS_DIM - N_SEM
SEM_OFF = NUM_EMB * N_SEM
TOTAL_EMB = SEM_OFF + MAX_TAG * N_TAG + 1
PAD_IDX = TOTAL_EMB - 1
B, L = 4096, 200
LF = 4

NC = 2
NS = 16
NW = NC * NS
LANES = 16

CHUNK = 512
IDX_ROWS = CHUNK // 128
NB = 2

SEQ_N = B * L
FUT_N = B * LF
SEQ_PER_W = SEQ_N // NW
FUT_PER_W = FUT_N // NW
SEQ_CHUNKS = SEQ_PER_W // CHUNK


def _compute_indices_chunk(sem_v, tok_v, idx_v):
    for i in range(CHUNK // LANES):
        s = sem_v[pl.ds(i * LANES, LANES)]
        t = tok_v[pl.ds(i * LANES, LANES)]
        sem_c = jnp.minimum(jnp.maximum(s, 0), NUM_EMB - 1)
        tag_c = jnp.minimum(jnp.maximum(s, 0), MAX_TAG - 1)
        idx_sem = t * NUM_EMB + sem_c
        tag_layer = t - N_SEM
        idx_tag = jnp.where(
            tag_layer < N_TAG, SEM_OFF + tag_layer * MAX_TAG + tag_c, PAD_IDX
        )
        idx = jnp.where(t < N_SEM, idx_sem, idx_tag)
        idx_v[i // 8, pl.ds((i % 8) * LANES, LANES)] = idx


def _body(sem_seq, tok_seq, sem_fut, tok_fut, table,
          out_seq, out_fut, sem_v, tok_v, idx_v, rows_v,
          id_sems, g_sems, w_sems):
    wid = lax.axis_index("s") * NC + lax.axis_index("c")
    seq_base0 = wid * SEQ_PER_W
    fut_base = wid * FUT_PER_W

    def fire_ids(src_s, src_t, base, b):
        pltpu.async_copy(src_s.at[pl.ds(base, CHUNK)], sem_v.at[b],
                         id_sems.at[b])
        pltpu.async_copy(src_t.at[pl.ds(base, CHUNK)], tok_v.at[b],
                         id_sems.at[b])

    def wait_ids(b):
        pltpu.make_async_copy(sem_seq.at[pl.ds(0, CHUNK)], sem_v.at[b],
                              id_sems.at[b]).wait()
        pltpu.make_async_copy(tok_seq.at[pl.ds(0, CHUNK)], tok_v.at[b],
                              id_sems.at[b]).wait()

    def fire_gathers(b):
        for j in range(IDX_ROWS):
            pltpu.async_copy(table.at[idx_v.at[b, j]],
                             rows_v.at[b, pl.ds(j * 128, 128)],
                             g_sems.at[b])

    def wait_gathers(b):
        for j in range(IDX_ROWS):
            pltpu.make_async_copy(table.at[idx_v.at[b, j]],
                                  rows_v.at[b, pl.ds(j * 128, 128)],
                                  g_sems.at[b]).wait()

    def fire_write(out_ref, base, b):
        pltpu.async_copy(rows_v.at[b], out_ref.at[pl.ds(base, CHUNK)],
                         w_sems.at[b])

    def wait_write(b):
        pltpu.make_async_copy(rows_v.at[b], out_seq.at[pl.ds(0, CHUNK)],
                              w_sems.at[b]).wait()

    def compute(b):
        _compute_indices_chunk(sem_v.at[b], tok_v.at[b], idx_v.at[b])

    sbase = lambda i: seq_base0 + i * CHUNK

    fire_ids(sem_seq, tok_seq, sbase(0), 0)
    fire_ids(sem_seq, tok_seq, sbase(1), 1)

    wait_ids(0)
    compute(0)
    fire_ids(sem_seq, tok_seq, sbase(2), 0)
    fire_gathers(0)

    wait_ids(1)
    compute(1)
    fire_ids(sem_seq, tok_seq, sbase(3), 1)
    fire_gathers(1)
    wait_gathers(0)
    fire_write(out_seq, sbase(0), 0)

    def loop_body(g, carry):
        for b in (0, 1):
            i = 2 * g + b
            nb = 1 - b
            wait_ids(b)
            compute(b)
            fire_ids(sem_seq, tok_seq, sbase(i + 2), b)
            wait_write(b)
            fire_gathers(b)
            wait_gathers(nb)
            fire_write(out_seq, sbase(i - 1), nb)
        return carry

    lax.fori_loop(1, SEQ_CHUNKS // 2 - 1, loop_body, 0)

    wait_ids(0)
    compute(0)
    fire_ids(sem_fut, tok_fut, fut_base, 0)
    wait_write(0)
    fire_gathers(0)
    wait_gathers(1)
    fire_write(out_seq, sbase(SEQ_CHUNKS - 3), 1)

    wait_ids(1)
    compute(1)
    wait_write(1)
    fire_gathers(1)
    wait_gathers(0)
    fire_write(out_seq, sbase(SEQ_CHUNKS - 2), 0)

    wait_ids(0)
    compute(0)
    wait_write(0)
    fire_gathers(0)
    wait_gathers(1)
    fire_write(out_seq, sbase(SEQ_CHUNKS - 1), 1)

    wait_gathers(0)
    fire_write(out_fut, fut_base, 0)
    wait_write(1)
    wait_write(0)


@jax.jit
def _emb_lookup(sem_seq, tok_seq, sem_fut, tok_fut, table):
    mesh = plsc.VectorSubcoreMesh(core_axis_name="c", subcore_axis_name="s")
    f = pl.kernel(
        _body,
        out_type=(
            jax.ShapeDtypeStruct((SEQ_N, EMB_DIM), jnp.bfloat16),
            jax.ShapeDtypeStruct((FUT_N, EMB_DIM), jnp.bfloat16),
        ),
        mesh=mesh,
        scratch_types=[
            pltpu.VMEM((NB, CHUNK), jnp.int32),
            pltpu.VMEM((NB, CHUNK), jnp.int32),
            pltpu.VMEM((NB, IDX_ROWS, 128), jnp.int32),
            pltpu.VMEM((NB, CHUNK, EMB_DIM), jnp.bfloat16),
            pltpu.SemaphoreType.DMA((NB,)),
            pltpu.SemaphoreType.DMA((NB,)),
            pltpu.SemaphoreType.DMA((NB,)),
        ],
        compiler_params=pltpu.CompilerParams(use_tc_tiling_on_sc=False),
    )
    return f(sem_seq, tok_seq, sem_fut, tok_fut, table)


def kernel(sem_ids, token_type_ids, sem_ids_fut, token_type_ids_fut, emb_table):
    out_seq, out_fut = _emb_lookup(
        sem_ids.reshape(-1),
        token_type_ids.reshape(-1),
        sem_ids_fut.reshape(-1),
        token_type_ids_fut.reshape(-1),
        emb_table.astype(jnp.bfloat16),
    )
    return (
        out_seq.astype(jnp.float32).reshape(B, L, EMB_DIM),
        out_fut.astype(jnp.float32).reshape(B, LF, EMB_DIM),
    )


# speedup vs baseline: 1.5828x; 1.0760x over previous
import functools

import jax
import jax.numpy as jnp
from jax import lax
from jax.experimental import pallas as pl
from jax.experimental.pallas import tpu as pltpu
from jax.experimental.pallas import tpu_sc as plsc

NUM_EMB = 100000
SEM_IDS_DIM = 4
EMB_DIM = 64
N_SEM = 3
MAX_TAG = 1000
N_TAG = SEM_IDS_DIM - N_SEM
SEM_OFF = NUM_EMB * N_SEM
TOTAL_EMB = SEM_OFF + MAX_TAG * N_TAG + 1
PAD_IDX = TOTAL_EMB - 1
B, L = 4096, 200
LF = 4

NC = 2
NS = 16
NW = NC * NS
LANES = 16

CHUNK = 512
IDX_ROWS = CHUNK // 128
NB = 2
CELEMS = CHUNK * EMB_DIM
CGROUPS = CELEMS // 32
CUNROLL = 8

SEQ_N = B * L
FUT_N = B * LF
SEQ_PER_W = SEQ_N // NW
FUT_PER_W = FUT_N // NW
SEQ_CHUNKS = SEQ_PER_W // CHUNK


def _compute_indices_chunk(sem_v, tok_v, idx_v):
    for i in range(CHUNK // LANES):
        s = sem_v[pl.ds(i * LANES, LANES)]
        t = tok_v[pl.ds(i * LANES, LANES)]
        sem_c = jnp.minimum(jnp.maximum(s, 0), NUM_EMB - 1)
        tag_c = jnp.minimum(jnp.maximum(s, 0), MAX_TAG - 1)
        idx_sem = t * NUM_EMB + sem_c
        tag_layer = t - N_SEM
        idx_tag = jnp.where(
            tag_layer < N_TAG, SEM_OFF + tag_layer * MAX_TAG + tag_c, PAD_IDX
        )
        idx = jnp.where(t < N_SEM, idx_sem, idx_tag)
        idx_v[i // 8, pl.ds((i % 8) * LANES, LANES)] = idx


def _body(sem_seq, tok_seq, sem_fut, tok_fut, table,
          out_seq, out_fut, sem_v, tok_v, idx_v, rows_bf, rows_f32,
          id_sems, g_sems, w_sems):
    wid = lax.axis_index("s") * NC + lax.axis_index("c")
    seq_base0 = wid * SEQ_PER_W
    fut_base = wid * FUT_PER_W
    ev2 = 2 * lax.iota(jnp.int32, 16)

    def fire_ids(src_s, src_t, base, b):
        pltpu.async_copy(src_s.at[pl.ds(base, CHUNK)], sem_v.at[b],
                         id_sems.at[b])
        pltpu.async_copy(src_t.at[pl.ds(base, CHUNK)], tok_v.at[b],
                         id_sems.at[b])

    def wait_ids(b):
        pltpu.make_async_copy(sem_seq.at[pl.ds(0, CHUNK)], sem_v.at[b],
                              id_sems.at[b]).wait()
        pltpu.make_async_copy(tok_seq.at[pl.ds(0, CHUNK)], tok_v.at[b],
                              id_sems.at[b]).wait()

    def fire_gathers(b):
        for j in range(IDX_ROWS):
            pltpu.async_copy(table.at[idx_v.at[b, j]],
                             rows_bf.at[b, pl.ds(j * 128, 128)],
                             g_sems.at[b])

    def wait_gathers(b):
        for j in range(IDX_ROWS):
            pltpu.make_async_copy(table.at[idx_v.at[b, j]],
                                  rows_bf.at[b, pl.ds(j * 128, 128)],
                                  g_sems.at[b]).wait()

    def convert(b):
        def conv_iter(ro, carry):
            for u in range(CUNROLL):
                r = ro * CUNROLL + u
                for half in (0, 1):
                    src = rows_bf[b, r, pl.ds(half * 32, 32)]
                    w = plsc.bitcast(src, jnp.int32)
                    lo = plsc.bitcast(w << 16, jnp.float32)
                    hi = plsc.bitcast(w & jnp.int32(-65536), jnp.float32)
                    pos = r * EMB_DIM + half * 32 + ev2
                    plsc.store_scatter(rows_f32.at[b], [pos], lo)
                    plsc.store_scatter(rows_f32.at[b], [pos + 1], hi)
            return carry

        lax.fori_loop(0, CHUNK // CUNROLL, conv_iter, 0)

    def fire_write(out_ref, base, b):
        pltpu.async_copy(rows_f32.at[b],
                         out_ref.at[pl.ds(base * EMB_DIM, CELEMS)],
                         w_sems.at[b])

    def wait_write(b):
        pltpu.make_async_copy(rows_f32.at[b], out_seq.at[pl.ds(0, CELEMS)],
                              w_sems.at[b]).wait()

    def compute(b):
        _compute_indices_chunk(sem_v.at[b], tok_v.at[b], idx_v.at[b])

    sbase = lambda i: seq_base0 + i * CHUNK

    fire_ids(sem_seq, tok_seq, sbase(0), 0)
    fire_ids(sem_seq, tok_seq, sbase(1), 1)

    wait_ids(0)
    compute(0)
    fire_ids(sem_seq, tok_seq, sbase(2), 0)
    fire_gathers(0)

    wait_ids(1)
    compute(1)
    fire_ids(sem_seq, tok_seq, sbase(3), 1)
    fire_gathers(1)
    wait_gathers(0)
    convert(0)
    fire_write(out_seq, sbase(0), 0)

    def loop_body(g, carry):
        for b in (0, 1):
            i = 2 * g + b
            nb = 1 - b
            wait_ids(b)
            compute(b)
            fire_ids(sem_seq, tok_seq, sbase(i + 2), b)
            wait_write(b)
            fire_gathers(b)
            wait_gathers(nb)
            convert(nb)
            fire_write(out_seq, sbase(i - 1), nb)
        return carry

    lax.fori_loop(1, SEQ_CHUNKS // 2 - 1, loop_body, 0)

    wait_ids(0)
    compute(0)
    fire_ids(sem_fut, tok_fut, fut_base, 0)
    wait_write(0)
    fire_gathers(0)
    wait_gathers(1)
    convert(1)
    fire_write(out_seq, sbase(SEQ_CHUNKS - 3), 1)

    wait_ids(1)
    compute(1)
    wait_write(1)
    fire_gathers(1)
    wait_gathers(0)
    convert(0)
    fire_write(out_seq, sbase(SEQ_CHUNKS - 2), 0)

    wait_ids(0)
    compute(0)
    wait_write(0)
    fire_gathers(0)
    wait_gathers(1)
    convert(1)
    fire_write(out_seq, sbase(SEQ_CHUNKS - 1), 1)

    wait_gathers(0)
    convert(0)
    fire_write(out_fut, fut_base, 0)
    wait_write(1)
    wait_write(0)


@jax.jit
def _emb_lookup(sem_seq, tok_seq, sem_fut, tok_fut, table):
    mesh = plsc.VectorSubcoreMesh(core_axis_name="c", subcore_axis_name="s")
    f = pl.kernel(
        _body,
        out_type=(
            jax.ShapeDtypeStruct((SEQ_N * EMB_DIM,), jnp.float32),
            jax.ShapeDtypeStruct((FUT_N * EMB_DIM,), jnp.float32),
        ),
        mesh=mesh,
        scratch_types=[
            pltpu.VMEM((NB, CHUNK), jnp.int32),
            pltpu.VMEM((NB, CHUNK), jnp.int32),
            pltpu.VMEM((NB, IDX_ROWS, 128), jnp.int32),
            pltpu.VMEM((NB, CHUNK, EMB_DIM), jnp.bfloat16),
            pltpu.VMEM((NB, CELEMS), jnp.float32),
            pltpu.SemaphoreType.DMA((NB,)),
            pltpu.SemaphoreType.DMA((NB,)),
            pltpu.SemaphoreType.DMA((NB,)),
        ],
        compiler_params=pltpu.CompilerParams(
            use_tc_tiling_on_sc=False, needs_layout_passes=False
        ),
    )
    return f(sem_seq, tok_seq, sem_fut, tok_fut, table)


def kernel(sem_ids, token_type_ids, sem_ids_fut, token_type_ids_fut, emb_table):
    out_seq, out_fut = _emb_lookup(
        sem_ids.reshape(-1),
        token_type_ids.reshape(-1),
        sem_ids_fut.reshape(-1),
        token_type_ids_fut.reshape(-1),
        emb_table.astype(jnp.bfloat16),
    )
    return (
        out_seq.reshape(B, L, EMB_DIM),
        out_fut.reshape(B, LF, EMB_DIM),
    )
